# baseline (speedup 1.0000x reference)
import functools

import jax
import jax.numpy as jnp
from jax.experimental import pallas as pl

NUM_GROUP = 512
GROUP_SIZE = 32
IN_CH = 3
ENC_CH = 384


def _fps(xyz, n_samples):
    B, N, _ = xyz.shape
    dists0 = jnp.full((B, N), 1e10, dtype=xyz.dtype)
    far0 = jnp.zeros((B,), dtype=jnp.int32)

    def step(carry, _):
        dists, farthest = carry
        centroid = jnp.take_along_axis(xyz, farthest[:, None, None].astype(jnp.int32), axis=1)
        d = jnp.sum((xyz - centroid) ** 2, axis=-1)
        dists = jnp.minimum(dists, d)
        nxt = jnp.argmax(dists, axis=1).astype(jnp.int32)
        return (dists, nxt), farthest

    (_, _), idxs = jax.lax.scan(step, (dists0, far0), None, length=n_samples)
    return jnp.transpose(idxs)


def _index_points(points, idx):
    return jax.vmap(lambda p, i: p[i])(points, idx)


def _square_distance(src, dst):
    d = -2.0 * jnp.einsum('bsc,bnc->bsn', src, dst)
    d = d + jnp.sum(src ** 2, -1)[:, :, None]
    d = d + jnp.sum(dst ** 2, -1)[:, None, :]
    return d


def _conv1(x, W, b):
    return jnp.einsum('oi,bik->bok', W, x) + b[None, :, None]


def _batchnorm(x, gamma, beta, eps=1e-5):
    mean = jnp.mean(x, axis=(0, 2), keepdims=True)
    var = jnp.var(x, axis=(0, 2), keepdims=True)
    xn = (x - mean) / jnp.sqrt(var + eps)
    return gamma[None, :, None] * xn + beta[None, :, None]


def _sub_kernel(nb_ref, c_ref, o_ref):
    o_ref[...] = nb_ref[...] - c_ref[...]


def _encoder(neighborhood, W1, b1, g1, be1, W2, b2, W3, b3, g3, be3, W4, b4):
    bs, g, n, _ = neighborhood.shape
    pg = neighborhood.reshape(bs * g, n, IN_CH).transpose(0, 2, 1)
    f = _conv1(pg, W1, b1)
    f = jax.nn.relu(_batchnorm(f, g1, be1))
    f = _conv1(f, W2, b2)
    fg = jnp.max(f, axis=2, keepdims=True)
    f = jnp.concatenate([jnp.broadcast_to(fg, (bs * g, 256, n)), f], axis=1)
    f = _conv1(f, W3, b3)
    f = jax.nn.relu(_batchnorm(f, g3, be3))
    f = _conv1(f, W4, b4)
    fg = jnp.max(f, axis=2)
    return fg.reshape(bs, g, ENC_CH)


def kernel(xyz, W1, b1, g1, be1, W2, b2, W3, b3, g3, be3, W4, b4):
    B, N, _ = xyz.shape
    c_idx = _fps(xyz, NUM_GROUP)
    center = _index_points(xyz, c_idx)
    dist = _square_distance(center, xyz)
    _, idx = jax.lax.top_k(-dist, GROUP_SIZE)
    return idx
    center = _index_points(xyz, c_idx)
    dist = _square_distance(center, xyz)
    _, idx = jax.lax.top_k(-dist, GROUP_SIZE)
    neighborhood = _index_points(xyz, idx)
    nb_flat = neighborhood.reshape(B * NUM_GROUP, GROUP_SIZE * 3)
    c_flat = jnp.tile(center.reshape(B * NUM_GROUP, 3), (1, GROUP_SIZE))
    nb_flat = pl.pallas_call(
        _sub_kernel,
        out_shape=jax.ShapeDtypeStruct((B * NUM_GROUP, GROUP_SIZE * 3), jnp.float32),
    )(nb_flat, c_flat)
    neighborhood = nb_flat.reshape(B, NUM_GROUP, GROUP_SIZE, 3)
    tokens = _encoder(neighborhood, W1, b1, g1, be1, W2, b2, W3, b3, g3, be3, W4, b4)
    return tokens


# speedup vs baseline: 1.4564x; 1.4564x over previous
import functools

import jax
import jax.numpy as jnp
from jax.experimental import pallas as pl
from jax.experimental.pallas import tpu as pltpu

NUM_GROUP = 512
GROUP_SIZE = 32
IN_CH = 3
ENC_CH = 384


def _fps(xyz, n_samples):
    B, N, _ = xyz.shape
    dists0 = jnp.full((B, N), 1e10, dtype=xyz.dtype)
    far0 = jnp.zeros((B,), dtype=jnp.int32)

    def step(carry, _):
        dists, farthest = carry
        centroid = jnp.take_along_axis(xyz, farthest[:, None, None].astype(jnp.int32), axis=1)
        d = jnp.sum((xyz - centroid) ** 2, axis=-1)
        dists = jnp.minimum(dists, d)
        nxt = jnp.argmax(dists, axis=1).astype(jnp.int32)
        return (dists, nxt), farthest

    (_, _), idxs = jax.lax.scan(step, (dists0, far0), None, length=n_samples)
    return jnp.transpose(idxs)


def _fps_kernel(xt_ref, center_ref, dists_ref):
    _, B, N = xt_ref.shape
    G = center_ref.shape[2]
    x = xt_ref[0]
    y = xt_ref[1]
    z = xt_ref[2]
    dists_ref[...] = jnp.full((B, N), 1e10, jnp.float32)
    iota_n = jax.lax.broadcasted_iota(jnp.int32, (B, N), 1)
    iota_g = jax.lax.broadcasted_iota(jnp.int32, (B, G), 1)

    def body(t, far):
        oh = iota_n == far
        cx = jnp.sum(jnp.where(oh, x, 0.0), axis=1, keepdims=True)
        cy = jnp.sum(jnp.where(oh, y, 0.0), axis=1, keepdims=True)
        cz = jnp.sum(jnp.where(oh, z, 0.0), axis=1, keepdims=True)
        ohg = iota_g == t
        center_ref[0] = jnp.where(ohg, cx, center_ref[0])
        center_ref[1] = jnp.where(ohg, cy, center_ref[1])
        center_ref[2] = jnp.where(ohg, cz, center_ref[2])
        dx = x - cx
        dy = y - cy
        dz = z - cz
        d = dx * dx + dy * dy + dz * dz
        nd = jnp.minimum(dists_ref[...], d)
        dists_ref[...] = nd
        m = jnp.max(nd, axis=1, keepdims=True)
        far2 = jnp.min(jnp.where(nd == m, iota_n, N), axis=1, keepdims=True)
        return far2

    far0 = jnp.zeros((B, 1), jnp.int32)
    jax.lax.fori_loop(0, G, body, far0)


def _fps_centers(xyz, n_samples, interpret=False):
    B, N, _ = xyz.shape
    xt = jnp.transpose(xyz, (2, 0, 1))
    center3 = pl.pallas_call(
        _fps_kernel,
        out_shape=jax.ShapeDtypeStruct((3, B, n_samples), jnp.float32),
        scratch_shapes=[pltpu.VMEM((B, N), jnp.float32)],
        interpret=interpret,
    )(xt)
    return jnp.transpose(center3, (1, 2, 0))


def _index_points(points, idx):
    return jax.vmap(lambda p, i: p[i])(points, idx)


def _square_distance(src, dst):
    d = -2.0 * jnp.einsum('bsc,bnc->bsn', src, dst)
    d = d + jnp.sum(src ** 2, -1)[:, :, None]
    d = d + jnp.sum(dst ** 2, -1)[:, None, :]
    return d


def _conv1(x, W, b):
    return jnp.einsum('oi,bik->bok', W, x) + b[None, :, None]


def _batchnorm(x, gamma, beta, eps=1e-5):
    mean = jnp.mean(x, axis=(0, 2), keepdims=True)
    var = jnp.var(x, axis=(0, 2), keepdims=True)
    xn = (x - mean) / jnp.sqrt(var + eps)
    return gamma[None, :, None] * xn + beta[None, :, None]


def _sub_kernel(nb_ref, c_ref, o_ref):
    o_ref[...] = nb_ref[...] - c_ref[...]


def _encoder(neighborhood, W1, b1, g1, be1, W2, b2, W3, b3, g3, be3, W4, b4):
    bs, g, n, _ = neighborhood.shape
    pg = neighborhood.reshape(bs * g, n, IN_CH).transpose(0, 2, 1)
    f = _conv1(pg, W1, b1)
    f = jax.nn.relu(_batchnorm(f, g1, be1))
    f = _conv1(f, W2, b2)
    fg = jnp.max(f, axis=2, keepdims=True)
    f = jnp.concatenate([jnp.broadcast_to(fg, (bs * g, 256, n)), f], axis=1)
    f = _conv1(f, W3, b3)
    f = jax.nn.relu(_batchnorm(f, g3, be3))
    f = _conv1(f, W4, b4)
    fg = jnp.max(f, axis=2)
    return fg.reshape(bs, g, ENC_CH)


def kernel(xyz, W1, b1, g1, be1, W2, b2, W3, b3, g3, be3, W4, b4):
    B, N, _ = xyz.shape
    center = _fps_centers(xyz, NUM_GROUP)
    dist = _square_distance(center, xyz)
    _, idx = jax.lax.top_k(-dist, GROUP_SIZE)
    neighborhood = _index_points(xyz, idx)
    nb_flat = neighborhood.reshape(B * NUM_GROUP, GROUP_SIZE * 3)
    c_flat = jnp.tile(center.reshape(B * NUM_GROUP, 3), (1, GROUP_SIZE))
    nb_flat = pl.pallas_call(
        _sub_kernel,
        out_shape=jax.ShapeDtypeStruct((B * NUM_GROUP, GROUP_SIZE * 3), jnp.float32),
    )(nb_flat, c_flat)
    neighborhood = nb_flat.reshape(B, NUM_GROUP, GROUP_SIZE, 3)
    tokens = _encoder(neighborhood, W1, b1, g1, be1, W2, b2, W3, b3, g3, be3, W4, b4)
    return tokens


# speedup vs baseline: 5.0225x; 3.4487x over previous
import functools

import jax
import jax.numpy as jnp
from jax.experimental import pallas as pl
from jax.experimental.pallas import tpu as pltpu

NUM_GROUP = 512
GROUP_SIZE = 32
IN_CH = 3
ENC_CH = 384


def _fps(xyz, n_samples):
    B, N, _ = xyz.shape
    dists0 = jnp.full((B, N), 1e10, dtype=xyz.dtype)
    far0 = jnp.zeros((B,), dtype=jnp.int32)

    def step(carry, _):
        dists, farthest = carry
        centroid = jnp.take_along_axis(xyz, farthest[:, None, None].astype(jnp.int32), axis=1)
        d = jnp.sum((xyz - centroid) ** 2, axis=-1)
        dists = jnp.minimum(dists, d)
        nxt = jnp.argmax(dists, axis=1).astype(jnp.int32)
        return (dists, nxt), farthest

    (_, _), idxs = jax.lax.scan(step, (dists0, far0), None, length=n_samples)
    return jnp.transpose(idxs)


def _fps_kernel(xt_ref, center_ref, dists_ref):
    _, B, N = xt_ref.shape
    G = center_ref.shape[2]
    x = xt_ref[0]
    y = xt_ref[1]
    z = xt_ref[2]
    dists_ref[...] = jnp.full((B, N), 1e10, jnp.float32)
    iota_n = jax.lax.broadcasted_iota(jnp.int32, (B, N), 1)
    iota_g = jax.lax.broadcasted_iota(jnp.int32, (B, G), 1)

    def body(t, far):
        oh = iota_n == far
        cx = jnp.sum(jnp.where(oh, x, 0.0), axis=1, keepdims=True)
        cy = jnp.sum(jnp.where(oh, y, 0.0), axis=1, keepdims=True)
        cz = jnp.sum(jnp.where(oh, z, 0.0), axis=1, keepdims=True)
        ohg = iota_g == t
        center_ref[0] = jnp.where(ohg, cx, center_ref[0])
        center_ref[1] = jnp.where(ohg, cy, center_ref[1])
        center_ref[2] = jnp.where(ohg, cz, center_ref[2])
        dx = x - cx
        dy = y - cy
        dz = z - cz
        d = dx * dx + dy * dy + dz * dz
        nd = jnp.minimum(dists_ref[...], d)
        dists_ref[...] = nd
        m = jnp.max(nd, axis=1, keepdims=True)
        far2 = jnp.min(jnp.where(nd == m, iota_n, N), axis=1, keepdims=True)
        return far2

    far0 = jnp.zeros((B, 1), jnp.int32)
    jax.lax.fori_loop(0, G, body, far0)


def _fps_centers(xyz, n_samples, interpret=False):
    B, N, _ = xyz.shape
    xt = jnp.transpose(xyz, (2, 0, 1))
    center3 = pl.pallas_call(
        _fps_kernel,
        out_shape=jax.ShapeDtypeStruct((3, B, n_samples), jnp.float32),
        scratch_shapes=[pltpu.VMEM((B, N), jnp.float32)],
        interpret=interpret,
    )(xt)
    return jnp.transpose(center3, (1, 2, 0))


_GBLK = 128


def _knn_kernel(din_ref, xt_ref, c_ref, nbh_ref, d_ref):
    N = din_ref.shape[2]
    K = nbh_ref.shape[1]
    c = c_ref[0]
    d_ref[...] = din_ref[0]
    iota_n = jax.lax.broadcasted_iota(jnp.int32, (c.shape[0], N), 1)
    x = xt_ref[0, 0:1]
    y = xt_ref[0, 1:2]
    z = xt_ref[0, 2:3]
    for k in range(K):
        dcur = d_ref[...]
        m = jnp.min(dcur, axis=1, keepdims=True)
        ii = jnp.min(jnp.where(dcur <= m, iota_n, N), axis=1, keepdims=True)
        sel = iota_n == ii
        gx = jnp.sum(jnp.where(sel, x, 0.0), axis=1, keepdims=True)
        gy = jnp.sum(jnp.where(sel, y, 0.0), axis=1, keepdims=True)
        gz = jnp.sum(jnp.where(sel, z, 0.0), axis=1, keepdims=True)
        nbk = jnp.concatenate([gx, gy, gz], axis=1)
        nbh_ref[0, k] = nbk - c
        d_ref[...] = jnp.where(sel, 1e30, dcur)


def _knn_neighborhood(xyz, center, interpret=False):
    B, N, _ = xyz.shape
    G = center.shape[1]
    K = GROUP_SIZE
    dist = _square_distance(center, xyz)
    xt = jnp.transpose(xyz, (0, 2, 1))
    nbh = pl.pallas_call(
        _knn_kernel,
        grid=(B, G // _GBLK),
        in_specs=[
            pl.BlockSpec((1, _GBLK, N), lambda b, g: (b, g, 0)),
            pl.BlockSpec((1, 3, N), lambda b, g: (b, 0, 0)),
            pl.BlockSpec((1, _GBLK, 3), lambda b, g: (b, g, 0)),
        ],
        out_specs=pl.BlockSpec((1, K, _GBLK, 3), lambda b, g: (b, 0, g, 0)),
        out_shape=jax.ShapeDtypeStruct((B, K, G, 3), jnp.float32),
        scratch_shapes=[pltpu.VMEM((_GBLK, N), jnp.float32)],
        compiler_params=pltpu.CompilerParams(
            dimension_semantics=("parallel", "parallel")),
        interpret=interpret,
    )(dist, xt, center)
    return jnp.transpose(nbh, (0, 2, 1, 3))


def _index_points(points, idx):
    return jax.vmap(lambda p, i: p[i])(points, idx)


def _square_distance(src, dst):
    d = -2.0 * jnp.einsum('bsc,bnc->bsn', src, dst)
    d = d + jnp.sum(src ** 2, -1)[:, :, None]
    d = d + jnp.sum(dst ** 2, -1)[:, None, :]
    return d


def _conv1(x, W, b):
    return jnp.einsum('oi,bik->bok', W, x) + b[None, :, None]


def _batchnorm(x, gamma, beta, eps=1e-5):
    mean = jnp.mean(x, axis=(0, 2), keepdims=True)
    var = jnp.var(x, axis=(0, 2), keepdims=True)
    xn = (x - mean) / jnp.sqrt(var + eps)
    return gamma[None, :, None] * xn + beta[None, :, None]


def _sub_kernel(nb_ref, c_ref, o_ref):
    o_ref[...] = nb_ref[...] - c_ref[...]


def _encoder(neighborhood, W1, b1, g1, be1, W2, b2, W3, b3, g3, be3, W4, b4):
    bs, g, n, _ = neighborhood.shape
    pg = neighborhood.reshape(bs * g, n, IN_CH).transpose(0, 2, 1)
    f = _conv1(pg, W1, b1)
    f = jax.nn.relu(_batchnorm(f, g1, be1))
    f = _conv1(f, W2, b2)
    fg = jnp.max(f, axis=2, keepdims=True)
    f = jnp.concatenate([jnp.broadcast_to(fg, (bs * g, 256, n)), f], axis=1)
    f = _conv1(f, W3, b3)
    f = jax.nn.relu(_batchnorm(f, g3, be3))
    f = _conv1(f, W4, b4)
    fg = jnp.max(f, axis=2)
    return fg.reshape(bs, g, ENC_CH)


def kernel(xyz, W1, b1, g1, be1, W2, b2, W3, b3, g3, be3, W4, b4):
    B, N, _ = xyz.shape
    center = _fps_centers(xyz, NUM_GROUP)
    neighborhood = _knn_neighborhood(xyz, center)
    tokens = _encoder(neighborhood, W1, b1, g1, be1, W2, b2, W3, b3, g3, be3, W4, b4)
    return tokens
